# baseline (speedup 1.0000x reference)
import functools

import jax
import jax.numpy as jnp
from jax import lax
from jax.experimental import pallas as pl
from jax.experimental.pallas import tpu as pltpu
from jax.experimental.pallas import tpu_sc as plsc
from jax.extend import core as jex_core
from jax.interpreters import mlir
from jax._src import core as _jcore
from jax._src import dispatch as _dispatch
from jaxlib.mlir import ir

_H = 8192
_W = 8192
_MAX_OPS = 35
_BR = 128
_NCH = _H // _BR


def _scal_body(params_hbm, scal_hbm, pbuf, obuf):
    cid = lax.axis_index("c")
    sid = lax.axis_index("s")
    wid = sid + cid

    @pl.when(wid == 0)
    def _():
        pltpu.sync_copy(params_hbm, pbuf)
        lane = lax.iota(jnp.int32, 16)
        lim = jnp.where(
            lane == 0,
            _MAX_OPS - 1,
            jnp.where((lane == 1) | (lane == 2), _H - 1, 0),
        )
        obuf[...] = jnp.clip(pbuf[...], 0, lim)
        pltpu.sync_copy(obuf, scal_hbm)


_clip_scalars = functools.partial(
    pl.kernel,
    out_type=jax.ShapeDtypeStruct((16,), jnp.int32),
    mesh=plsc.VectorSubcoreMesh(core_axis_name="c", subcore_axis_name="s",
                                num_cores=1),
    scratch_types=[
        pltpu.VMEM((16,), jnp.int32),
        pltpu.VMEM((16,), jnp.int32),
    ],
)(_scal_body)


def _mask_body(scal_ref, out_hbm, zbuf, obuf, sem):
    r = jnp.clip(scal_ref[1], 0, _H - 1)
    c = jnp.clip(scal_ref[2], 0, _W - 1)

    rit = r % 8
    cit = c % 128
    rb = pl.multiple_of((r // 8) * 8, 8)
    cb = pl.multiple_of((c // 128) * 128, 128)

    zbuf[...] = jnp.zeros((_BR, _W), jnp.int8)
    ri = lax.broadcasted_iota(jnp.int32, (8, 128), 0)
    ci = lax.broadcasted_iota(jnp.int32, (8, 128), 1)
    obuf[...] = ((ri == rit) & (ci == cit)).astype(jnp.int8)

    copies = [
        pltpu.async_copy(zbuf, out_hbm.at[pl.ds(k * _BR, _BR), :], sem)
        for k in range(_NCH)
    ]
    for cp in copies:
        cp.wait()
    pltpu.async_copy(obuf, out_hbm.at[pl.ds(rb, 8), pl.ds(cb, 128)], sem).wait()


_fill_i8 = pl.pallas_call(
    _mask_body,
    in_specs=[pl.BlockSpec(memory_space=pltpu.SMEM)],
    out_specs=pl.BlockSpec(memory_space=pl.ANY),
    out_shape=jax.ShapeDtypeStruct((_H, _W), jnp.int8),
    scratch_shapes=[
        pltpu.VMEM((_BR, _W), jnp.int8),
        pltpu.VMEM((8, 128), jnp.int8),
        pltpu.SemaphoreType.DMA,
    ],
)

_pred_fill_p = jex_core.Primitive("pred_point_fill")
_pred_fill_p.def_abstract_eval(
    lambda p: _jcore.ShapedArray((_H, _W), jnp.bool_)
)
_pred_fill_p.def_impl(
    functools.partial(_dispatch.apply_primitive, _pred_fill_p)
)


def _pred_fill_lowering(ctx, params):
    int8_aval = _jcore.ShapedArray((_H, _W), jnp.int8)
    ctx8 = ctx.replace(avals_out=(int8_aval,))
    out = mlir.lower_fun(_fill_i8, multiple_results=False)(ctx8, params)
    (res,) = out if isinstance(out, (list, tuple)) else (out,)
    while isinstance(res, (list, tuple)):
        res = res[0]
    op = res.owner
    attrs = {}
    for i in range(len(op.attributes)):
        named = op.attributes[i]
        attrs[named.name] = named.attr
    pred_ty = ir.RankedTensorType.get(
        (_H, _W), ir.IntegerType.get_signless(1)
    )
    new_op = ir.Operation.create(
        op.name,
        results=[pred_ty],
        operands=list(op.operands),
        attributes=attrs,
    )
    op.erase()
    return [new_op.results[0]]


mlir.register_lowering(_pred_fill_p, _pred_fill_lowering)


def kernel(operation, action_type, row, col, grid_height, grid_width):
    head = jnp.stack(
        [
            jnp.asarray(operation, jnp.int32),
            jnp.asarray(row, jnp.int32),
            jnp.asarray(col, jnp.int32),
            jnp.asarray(action_type, jnp.int32),
        ]
    )
    params = jnp.concatenate([head, jnp.zeros((12,), jnp.int32)])
    scal = _clip_scalars(params)
    mask = _pred_fill_p.bind(params)
    return (mask, scal[0], scal[3], scal[1], scal[2])


# speedup vs baseline: 1.0072x; 1.0072x over previous
import functools

import jax
import jax.numpy as jnp
from jax import lax
from jax.experimental import pallas as pl
from jax.experimental.pallas import tpu as pltpu
from jax.experimental.pallas import tpu_sc as plsc
from jax.extend import core as jex_core
from jax.interpreters import mlir
from jax._src import core as _jcore
from jax._src import dispatch as _dispatch
from jaxlib.mlir import ir

_H = 8192
_W = 8192
_MAX_OPS = 35
_BR = 64
_NCH = _H // _BR


def _scal_body(params_hbm, scal_hbm, pbuf, obuf):
    cid = lax.axis_index("c")
    sid = lax.axis_index("s")
    wid = sid + cid

    @pl.when(wid == 0)
    def _():
        pltpu.sync_copy(params_hbm, pbuf)
        lane = lax.iota(jnp.int32, 16)
        lim = jnp.where(
            lane == 0,
            _MAX_OPS - 1,
            jnp.where((lane == 1) | (lane == 2), _H - 1, 0),
        )
        obuf[...] = jnp.clip(pbuf[...], 0, lim)
        pltpu.sync_copy(obuf, scal_hbm)


_clip_scalars = functools.partial(
    pl.kernel,
    out_type=jax.ShapeDtypeStruct((16,), jnp.int32),
    mesh=plsc.VectorSubcoreMesh(core_axis_name="c", subcore_axis_name="s",
                                num_cores=1),
    scratch_types=[
        pltpu.VMEM((16,), jnp.int32),
        pltpu.VMEM((16,), jnp.int32),
    ],
)(_scal_body)


def _mask_body(scal_ref, out_hbm, zbuf, obuf, sem):
    r = jnp.clip(scal_ref[1], 0, _H - 1)
    c = jnp.clip(scal_ref[2], 0, _W - 1)

    rit = r % 8
    cit = c % 128
    rb = pl.multiple_of((r // 8) * 8, 8)
    cb = pl.multiple_of((c // 128) * 128, 128)

    zbuf[...] = jnp.zeros((_BR, _W), jnp.int8)
    ri = lax.broadcasted_iota(jnp.int32, (8, 128), 0)
    ci = lax.broadcasted_iota(jnp.int32, (8, 128), 1)
    obuf[...] = ((ri == rit) & (ci == cit)).astype(jnp.int8)

    copies = [
        pltpu.async_copy(zbuf, out_hbm.at[pl.ds(k * _BR, _BR), :], sem)
        for k in range(_NCH)
    ]
    for cp in copies:
        cp.wait()
    pltpu.async_copy(obuf, out_hbm.at[pl.ds(rb, 8), pl.ds(cb, 128)], sem).wait()


_fill_i8 = pl.pallas_call(
    _mask_body,
    in_specs=[pl.BlockSpec(memory_space=pltpu.SMEM)],
    out_specs=pl.BlockSpec(memory_space=pl.ANY),
    out_shape=jax.ShapeDtypeStruct((_H, _W), jnp.int8),
    scratch_shapes=[
        pltpu.VMEM((_BR, _W), jnp.int8),
        pltpu.VMEM((8, 128), jnp.int8),
        pltpu.SemaphoreType.DMA,
    ],
)

_pred_fill_p = jex_core.Primitive("pred_point_fill")
_pred_fill_p.def_abstract_eval(
    lambda p: _jcore.ShapedArray((_H, _W), jnp.bool_)
)
_pred_fill_p.def_impl(
    functools.partial(_dispatch.apply_primitive, _pred_fill_p)
)


def _pred_fill_lowering(ctx, params):
    int8_aval = _jcore.ShapedArray((_H, _W), jnp.int8)
    ctx8 = ctx.replace(avals_out=(int8_aval,))
    out = mlir.lower_fun(_fill_i8, multiple_results=False)(ctx8, params)
    (res,) = out if isinstance(out, (list, tuple)) else (out,)
    while isinstance(res, (list, tuple)):
        res = res[0]
    op = res.owner
    attrs = {}
    for i in range(len(op.attributes)):
        named = op.attributes[i]
        attrs[named.name] = named.attr
    pred_ty = ir.RankedTensorType.get(
        (_H, _W), ir.IntegerType.get_signless(1)
    )
    new_op = ir.Operation.create(
        op.name,
        results=[pred_ty],
        operands=list(op.operands),
        attributes=attrs,
    )
    op.erase()
    return [new_op.results[0]]


mlir.register_lowering(_pred_fill_p, _pred_fill_lowering)


def kernel(operation, action_type, row, col, grid_height, grid_width):
    head = jnp.stack(
        [
            jnp.asarray(operation, jnp.int32),
            jnp.asarray(row, jnp.int32),
            jnp.asarray(col, jnp.int32),
            jnp.asarray(action_type, jnp.int32),
        ]
    )
    params = jnp.concatenate([head, jnp.zeros((12,), jnp.int32)])
    scal = _clip_scalars(params)
    mask = _pred_fill_p.bind(params)
    return (mask, scal[0], scal[3], scal[1], scal[2])
